# Initial kernel scaffold; baseline (speedup 1.0000x reference)
import jax
import jax.numpy as jnp
from jax.experimental import pallas as pl


def kernel(parent_feature, Wp, bp, We, be, Ws, bs, Wel, bel, Wee, bee, Wne, bne, Wc, bc, Wc2, bc2):
    raise NotImplementedError("write your pallas kernel here")


# speedup vs baseline: 4.8841x; 4.8841x over previous
import jax
import jax.numpy as jnp
from jax.experimental import pallas as pl

B = 1
NF = 256
H = 256
C = 64
NI = 2
ET = 4
NS = 57
ETP = 8
NSP = 64
CC = C * C


def _pf_kernel(parent_ref, wpt_ref, bp_ref, out_ref):
    out_ref[...] = jax.nn.relu(
        jnp.dot(parent_ref[...], wpt_ref[...],
                preferred_element_type=jnp.float32) + bp_ref[...])


def _main_kernel(cf0_ref, wet_ref, bep_ref, wela_ref, welb_ref, bel_ref,
                 weet_ref, beep_ref, w1t_ref, w2t_ref, w3t_ref, w4r_ref,
                 bne_ref, wct_ref, bc_ref, wst_ref, bsp_ref, wc2t_ref,
                 bc2_ref, outf_ref, sem_ref, cel_ref, eel_ref):
    relu = jax.nn.relu
    f32 = jnp.float32
    cf0 = cf0_ref[...]

    cel = jnp.dot(cf0, wet_ref[...], preferred_element_type=f32) + bep_ref[...]
    cel_ref[...] = cel
    exists = cel[:, 0:1] > 0.0

    P = jnp.dot(cf0, wela_ref[...], preferred_element_type=f32) + bel_ref[...]
    Q = jnp.dot(cf0, welb_ref[...], preferred_element_type=f32)
    el3 = relu(P[:, None, :] + Q[None, :, :])
    el2 = el3.reshape(CC, H)

    eel = jnp.dot(el2, weet_ref[...], preferred_element_type=f32) + beep_ref[...]
    eel_ref[...] = eel

    ex2 = (exists[:, None, :] & exists[None, :, :]).reshape(CC, 1)
    em = (eel > 0.0) & ex2
    has_edges = jnp.any(em)
    mf = em.astype(f32)

    feats = cf0
    iter_feats = [feats]
    for k in range(NI):
        A = jnp.dot(feats, w1t_ref[k], preferred_element_type=f32) \
            + bne_ref[k:k + 1, :]
        Bm = jnp.dot(feats, w2t_ref[k], preferred_element_type=f32)
        E = jnp.dot(el2, w3t_ref[k], preferred_element_type=f32)
        base = (A[:, None, :] + Bm[None, :, :]).reshape(CC, H) + E
        s = jnp.zeros((CC, H), dtype=f32)
        for e in range(ET):
            t = base + eel[:, e:e + 1] * w4r_ref[k * ET + e:k * ET + e + 1, :]
            s = s + relu(t) * mf[:, e:e + 1]
        seg = s.reshape(C, C, H).sum(axis=1)
        feats = jnp.where(has_edges, seg, feats)
        iter_feats.append(feats)

    cfcat = jnp.concatenate(iter_feats, axis=1)
    cfin = relu(jnp.dot(cfcat, wct_ref[...], preferred_element_type=f32)
                + bc_ref[...])
    sem_ref[...] = jnp.dot(cfin, wst_ref[...], preferred_element_type=f32) \
        + bsp_ref[...]
    outf_ref[...] = relu(jnp.dot(cfin, wc2t_ref[...],
                                 preferred_element_type=f32) + bc2_ref[...])


def kernel(parent_feature, Wp, bp, We, be, Ws, bs, Wel, bel, Wee, bee,
           Wne, bne, Wc, bc, Wc2, bc2):
    f32 = jnp.float32

    BLK = 2048
    nblk = (H * C) // BLK
    pf = pl.pallas_call(
        _pf_kernel,
        grid=(nblk,),
        in_specs=[
            pl.BlockSpec((1, NF), lambda i: (0, 0)),
            pl.BlockSpec((NF, BLK), lambda i: (0, i)),
            pl.BlockSpec((1, BLK), lambda i: (0, i)),
        ],
        out_specs=pl.BlockSpec((1, BLK), lambda i: (0, i)),
        out_shape=jax.ShapeDtypeStruct((1, H * C), f32),
    )(parent_feature, Wp.T, bp.reshape(1, H * C))
    cf0 = pf.reshape(C, H)

    wet = jnp.zeros((H, ETP), f32).at[:, 0:1].set(We.T)
    bep = jnp.zeros((1, ETP), f32).at[0, 0].set(be[0])
    wela = Wel[:, :H].T
    welb = Wel[:, H:].T
    bel_r = bel.reshape(1, H)
    weet = jnp.zeros((H, ETP), f32).at[:, :ET].set(Wee.T)
    beep = jnp.zeros((1, ETP), f32).at[0, :ET].set(bee)
    w1t = Wne[:, :, :H].transpose(0, 2, 1)
    w2t = Wne[:, :, H:2 * H].transpose(0, 2, 1)
    w3t = Wne[:, :, 2 * H:3 * H].transpose(0, 2, 1)
    w4r = Wne[:, :, 3 * H:].transpose(0, 2, 1).reshape(NI * ET, H)
    wct = Wc.T
    bc_r = bc.reshape(1, H)
    wst = jnp.zeros((H, NSP), f32).at[:, :NS].set(Ws.T)
    bsp = jnp.zeros((1, NSP), f32).at[0, :NS].set(bs)
    wc2t = Wc2.T
    bc2_r = bc2.reshape(1, NF)

    outf, sem, cel, eel = pl.pallas_call(
        _main_kernel,
        out_shape=(
            jax.ShapeDtypeStruct((C, NF), f32),
            jax.ShapeDtypeStruct((C, NSP), f32),
            jax.ShapeDtypeStruct((C, ETP), f32),
            jax.ShapeDtypeStruct((CC, ETP), f32),
        ),
    )(cf0, wet, bep, wela, welb, bel_r, weet, beep, w1t, w2t, w3t, w4r,
      bne, wct, bc_r, wst, bsp, wc2t, bc2_r)

    out_feats = outf.reshape(B, C, NF)
    child_sem_logits = sem[:, :NS].reshape(B, C, NS)
    child_exists_logits = cel[:, 0:1].reshape(B, C, 1)
    edge_exists_logits = eel[:, :ET].reshape(B, C, C, ET)
    return (out_feats, child_sem_logits, child_exists_logits,
            edge_exists_logits)


# speedup vs baseline: 7.1457x; 1.4631x over previous
import jax
import jax.numpy as jnp
from jax.experimental import pallas as pl

B = 1
NF = 256
H = 256
C = 64
NI = 2
ET = 4
NS = 57
ETP = 8
NSP = 64
CC = C * C

_DNT = (((1,), (1,)), ((), ()))


def _dott(x, w):
    return jax.lax.dot_general(x, w, _DNT, preferred_element_type=jnp.float32)


def _pf_kernel(parent_ref, wp_ref, bp_ref, out_ref):
    out_ref[...] = jax.nn.relu(
        _dott(parent_ref[...], wp_ref[...]) + bp_ref[...])


def _main_kernel(cf0_ref, wep_ref, bep_ref, wel_ref, bel_ref,
                 weep_ref, beep_ref, wne_ref, w4r_ref,
                 bne_ref, wc_ref, bc_ref, wsp_ref, bsp_ref, wc2_ref,
                 bc2_ref, outf_ref, sem_ref, cel_ref, eel_ref):
    relu = jax.nn.relu
    f32 = jnp.float32
    cf0 = cf0_ref[...]

    cel = _dott(cf0, wep_ref[...]) + bep_ref[...]
    cel_ref[...] = cel
    exists = cel[:, 0:1] > 0.0

    P = _dott(cf0, wel_ref[:, :H]) + bel_ref[...]
    Q = _dott(cf0, wel_ref[:, H:])
    el3 = relu(P[:, None, :] + Q[None, :, :])
    el2 = el3.reshape(CC, H)

    eel = _dott(el2, weep_ref[...]) + beep_ref[...]
    eel_ref[...] = eel

    ex2 = (exists[:, None, :] & exists[None, :, :]).reshape(CC, 1)
    em = (eel > 0.0) & ex2
    has_edges = jnp.any(em)
    mf = em.astype(f32)

    feats = cf0
    iter_feats = [feats]
    for k in range(NI):
        wk = wne_ref[k]
        A = _dott(feats, wk[:, :H]) + bne_ref[k:k + 1, :]
        Bm = _dott(feats, wk[:, H:2 * H])
        E = _dott(el2, wk[:, 2 * H:3 * H])
        base = (A[:, None, :] + Bm[None, :, :]).reshape(CC, H) + E
        s = jnp.zeros((CC, H), dtype=f32)
        for e in range(ET):
            t = base + eel[:, e:e + 1] * w4r_ref[k * ET + e:k * ET + e + 1, :]
            s = s + relu(t) * mf[:, e:e + 1]
        seg = s.reshape(C, C, H).sum(axis=1)
        feats = jnp.where(has_edges, seg, feats)
        iter_feats.append(feats)

    cfcat = jnp.concatenate(iter_feats, axis=1)
    cfin = relu(_dott(cfcat, wc_ref[...]) + bc_ref[...])
    sem_ref[...] = _dott(cfin, wsp_ref[...]) + bsp_ref[...]
    outf_ref[...] = relu(_dott(cfin, wc2_ref[...]) + bc2_ref[...])


def kernel(parent_feature, Wp, bp, We, be, Ws, bs, Wel, bel, Wee, bee,
           Wne, bne, Wc, bc, Wc2, bc2):
    f32 = jnp.float32

    BLKR = 2048
    nblk = (H * C) // BLKR
    pf = pl.pallas_call(
        _pf_kernel,
        grid=(nblk,),
        in_specs=[
            pl.BlockSpec((1, NF), lambda i: (0, 0)),
            pl.BlockSpec((BLKR, NF), lambda i: (i, 0)),
            pl.BlockSpec((1, BLKR), lambda i: (0, i)),
        ],
        out_specs=pl.BlockSpec((1, BLKR), lambda i: (0, i)),
        out_shape=jax.ShapeDtypeStruct((1, H * C), f32),
    )(parent_feature, Wp, bp.reshape(1, H * C))
    cf0 = pf.reshape(C, H)

    wep = jnp.zeros((ETP, H), f32).at[0:1, :].set(We)
    bep = jnp.zeros((1, ETP), f32).at[0, 0].set(be[0])
    bel_r = bel.reshape(1, H)
    weep = jnp.zeros((ETP, H), f32).at[:ET, :].set(Wee)
    beep = jnp.zeros((1, ETP), f32).at[0, :ET].set(bee)
    w4r = Wne[:, :, 3 * H:].transpose(0, 2, 1).reshape(NI * ET, H)
    bc_r = bc.reshape(1, H)
    wsp = jnp.zeros((NSP, H), f32).at[:NS, :].set(Ws)
    bsp = jnp.zeros((1, NSP), f32).at[0, :NS].set(bs)
    bc2_r = bc2.reshape(1, NF)

    outf, sem, cel, eel = pl.pallas_call(
        _main_kernel,
        out_shape=(
            jax.ShapeDtypeStruct((C, NF), f32),
            jax.ShapeDtypeStruct((C, NSP), f32),
            jax.ShapeDtypeStruct((C, ETP), f32),
            jax.ShapeDtypeStruct((CC, ETP), f32),
        ),
    )(cf0, wep, bep, Wel, bel_r, weep, beep, Wne, w4r,
      bne, Wc, bc_r, wsp, bsp, Wc2, bc2_r)

    out_feats = outf.reshape(B, C, NF)
    child_sem_logits = sem[:, :NS].reshape(B, C, NS)
    child_exists_logits = cel[:, 0:1].reshape(B, C, 1)
    edge_exists_logits = eel[:, :ET].reshape(B, C, C, ET)
    return (out_feats, child_sem_logits, child_exists_logits,
            edge_exists_logits)


# speedup vs baseline: 8.9452x; 1.2518x over previous
import jax
import jax.numpy as jnp
from jax.experimental import pallas as pl

B = 1
NF = 256
H = 256
C = 64
NI = 2
ET = 4
NS = 57
CC = C * C

_DNT = (((1,), (1,)), ((), ()))


def _dott(x, w):
    return jax.lax.dot_general(x, w, _DNT, preferred_element_type=jnp.float32)


def _pf_kernel(parent_ref, wp_ref, bp_ref, out_ref):
    out_ref[...] = jax.nn.relu(
        _dott(parent_ref[...], wp_ref[...]) + bp_ref[...])


def _main_kernel(cf0_ref, we_ref, be_ref, wel_ref, bel_ref,
                 wee_ref, bee_ref, wne_ref, w4r_ref,
                 bne_ref, wc_ref, bc_ref, ws_ref, bs_ref, wc2_ref,
                 bc2_ref, outf_ref, sem_ref, cel_ref, eel_ref):
    relu = jax.nn.relu
    f32 = jnp.float32
    cf0 = cf0_ref[...]

    cel = jnp.sum(cf0 * we_ref[...], axis=1, keepdims=True) + be_ref[0, 0]
    cel_ref[...] = cel
    exists = cel > 0.0

    P = _dott(cf0, wel_ref[:, :H]) + bel_ref[...]
    Q = _dott(cf0, wel_ref[:, H:])
    el3 = relu(P[:, None, :] + Q[None, :, :])
    el2 = el3.reshape(CC, H)

    eel = _dott(el2, wee_ref[...]) + bee_ref[...]
    eel_ref[...] = eel

    ex2 = (exists[:, None, :] & exists[None, :, :]).reshape(CC, 1)
    em = (eel > 0.0) & ex2
    has_edges = jnp.any(em)
    mf = em.astype(f32)

    feats = cf0
    iter_feats = [feats]
    for k in range(NI):
        wk = wne_ref[k]
        A = _dott(feats, wk[:, :H]) + bne_ref[k:k + 1, :]
        Bm = _dott(feats, wk[:, H:2 * H])
        E = _dott(el2, wk[:, 2 * H:3 * H])
        base = (A[:, None, :] + Bm[None, :, :]).reshape(CC, H) + E
        s = jnp.zeros((CC, H), dtype=f32)
        for e in range(ET):
            t = base + eel[:, e:e + 1] * w4r_ref[k * ET + e:k * ET + e + 1, :]
            s = s + relu(t) * mf[:, e:e + 1]
        seg = s.reshape(C, C, H).sum(axis=1)
        feats = jnp.where(has_edges, seg, feats)
        iter_feats.append(feats)

    cfcat = jnp.concatenate(iter_feats, axis=1)
    cfin = relu(_dott(cfcat, wc_ref[...]) + bc_ref[...])
    sem_ref[...] = _dott(cfin, ws_ref[...]) + bs_ref[...]
    outf_ref[...] = relu(_dott(cfin, wc2_ref[...]) + bc2_ref[...])


def kernel(parent_feature, Wp, bp, We, be, Ws, bs, Wel, bel, Wee, bee,
           Wne, bne, Wc, bc, Wc2, bc2):
    f32 = jnp.float32

    BLKR = 2048
    nblk = (H * C) // BLKR
    pf = pl.pallas_call(
        _pf_kernel,
        grid=(nblk,),
        in_specs=[
            pl.BlockSpec((1, NF), lambda i: (0, 0)),
            pl.BlockSpec((BLKR, NF), lambda i: (i, 0)),
            pl.BlockSpec((1, BLKR), lambda i: (0, i)),
        ],
        out_specs=pl.BlockSpec((1, BLKR), lambda i: (0, i)),
        out_shape=jax.ShapeDtypeStruct((1, H * C), f32),
    )(parent_feature, Wp, bp.reshape(1, H * C))
    cf0 = pf.reshape(C, H)

    w4r = Wne[:, :, 3 * H:].transpose(0, 2, 1).reshape(NI * ET, H)

    outf, sem, cel, eel = pl.pallas_call(
        _main_kernel,
        out_shape=(
            jax.ShapeDtypeStruct((C, NF), f32),
            jax.ShapeDtypeStruct((C, NS), f32),
            jax.ShapeDtypeStruct((C, 1), f32),
            jax.ShapeDtypeStruct((CC, ET), f32),
        ),
    )(cf0, We, be.reshape(1, 1), Wel, bel.reshape(1, H), Wee,
      bee.reshape(1, ET), Wne, w4r, bne, Wc, bc.reshape(1, H), Ws,
      bs.reshape(1, NS), Wc2, bc2.reshape(1, NF))

    return (outf.reshape(B, C, NF), sem.reshape(B, C, NS),
            cel.reshape(B, C, 1), eel.reshape(B, C, C, ET))
